# baseline (speedup 1.0000x reference)
import numpy as np
import jax
import jax.numpy as jnp
from jax.experimental import pallas as pl
from jax.experimental.pallas import tpu as pltpu


def _probe_kernel(x_ref, o1_ref, o2_ref):
    v = x_ref[0, 0, 0]
    o1_ref[...] = jnp.full(o1_ref.shape, v, o1_ref.dtype)
    o2_ref[...] = jnp.full(o2_ref.shape, v, o2_ref.dtype)


def kernel(w, gamma, beta, mean, var, hidden_states):
    B, P, C = hidden_states.shape
    K = w.shape[0]
    kt = 64
    out_shapes = (
        jax.ShapeDtypeStruct((B, K // 2, 392, 128), hidden_states.dtype),
        jax.ShapeDtypeStruct((B, K // 2, 392, 128), hidden_states.dtype),
    )
    o1, o2 = pl.pallas_call(
        _probe_kernel,
        out_shape=out_shapes,
        grid=(B,),
        in_specs=[pl.BlockSpec((1, P, C), lambda b: (b, 0, 0))],
        out_specs=[
            pl.BlockSpec((1, kt, 392, 128), lambda b: (b, 0, 0, 0)),
            pl.BlockSpec((1, kt, 392, 128), lambda b: (b, 0, 0, 0)),
        ],
        compiler_params=pltpu.CompilerParams(
            dimension_semantics=("parallel",)),
    )(hidden_states)
    out = jnp.concatenate([o1, o2], axis=1)
    return out.reshape(B, K, 224, 224)


# speedup vs baseline: 1.1724x; 1.1724x over previous
import numpy as np
import jax
import jax.numpy as jnp
from jax import lax
from jax.experimental import pallas as pl
from jax.experimental.pallas import tpu as pltpu

_BN_EPS = 1e-5
_NBUF = 4


def _interp_matrix(n_in, n_out):
    m = np.zeros((n_out, n_in), dtype=np.float32)
    if n_in == 1:
        m[:, 0] = 1.0
        return m
    src = np.arange(n_out, dtype=np.float64) * (n_in - 1) / (n_out - 1)
    lo = np.clip(np.floor(src).astype(np.int64), 0, n_in - 2)
    frac = (src - lo).astype(np.float32)
    m[np.arange(n_out), lo] = 1.0 - frac
    m[np.arange(n_out), lo + 1] = frac
    return m


def _make_decoder_kernel(G, kt, num_steps):
    def _kernel(x_ref, w_ref, b_ref, mh_ref, mwT_ref, o_ref,
                u_ref, obuf_ref, sems):
        h = mh_ref.shape[1]
        i = pl.program_id(0)
        slot = lax.rem(i, _NBUF)
        b = lax.div(i, G)
        g = lax.rem(i, G)

        @pl.when(i >= _NBUF)
        def _():
            pltpu.make_async_copy(
                obuf_ref.at[slot], obuf_ref.at[slot], sems.at[slot]).wait()

        zt = lax.dot_general(
            w_ref[0], x_ref[0], (((1,), (1,)), ((), ())),
            preferred_element_type=jnp.float32)
        zt = jnp.maximum(zt + b_ref[0], 0.0)
        mwT = mwT_ref[...]
        for hh in range(h):
            u_ref[:, hh, :] = jnp.dot(
                zt[:, hh * h:(hh + 1) * h], mwT,
                preferred_element_type=jnp.float32)
        mh = mh_ref[...]
        kt_ = obuf_ref.shape[1]
        for c in range(kt_):
            obuf_ref[slot, c] = jnp.dot(
                mh, u_ref[c], preferred_element_type=jnp.float32
            ).astype(o_ref.dtype)

        pltpu.make_async_copy(
            obuf_ref.at[slot],
            o_ref.at[b, pl.ds(g * kt, kt)],
            sems.at[slot]).start()

        @pl.when(i == num_steps - 1)
        def _():
            for s in range(min(_NBUF, num_steps)):
                pltpu.make_async_copy(
                    obuf_ref.at[s], obuf_ref.at[s], sems.at[s]).wait()

    return _kernel


def _choose_kt(K):
    for t in (32, 16, 8, 4, 2, 1):
        if K % t == 0:
            return t
    return K


def kernel(w, gamma, beta, mean, var, hidden_states):
    B, P, C = hidden_states.shape
    h = int(round(np.sqrt(P)))
    assert h * h == P
    K = w.shape[0]
    OH, OW = 224, 224

    kt = _choose_kt(K)
    G = K // kt
    num_steps = B * G
    scale = gamma / jnp.sqrt(var + _BN_EPS)
    wf = (w * scale[:, None]).astype(jnp.float32)
    w_tiles = wf.reshape(G, kt, C)
    bias = (beta - mean * scale).reshape(G, kt, 1).astype(jnp.float32)
    mh = jnp.asarray(_interp_matrix(h, OH))
    mwT = jnp.asarray(_interp_matrix(h, OW).T)

    return pl.pallas_call(
        _make_decoder_kernel(G, kt, num_steps),
        out_shape=jax.ShapeDtypeStruct((B, K, OH, OW), hidden_states.dtype),
        grid=(num_steps,),
        in_specs=[
            pl.BlockSpec((1, P, C), lambda i: (i // G, 0, 0)),
            pl.BlockSpec((1, kt, C), lambda i: (i % G, 0, 0)),
            pl.BlockSpec((1, kt, 1), lambda i: (i % G, 0, 0)),
            pl.BlockSpec((OH, h), lambda i: (0, 0)),
            pl.BlockSpec((h, OW), lambda i: (0, 0)),
        ],
        out_specs=pl.BlockSpec(memory_space=pl.ANY),
        scratch_shapes=[
            pltpu.VMEM((kt, h, OW), jnp.float32),
            pltpu.VMEM((_NBUF, kt, OH, OW), jnp.float32),
            pltpu.SemaphoreType.DMA((_NBUF,)),
        ],
        compiler_params=pltpu.CompilerParams(
            dimension_semantics=("arbitrary",),
            vmem_limit_bytes=60 * 1024 * 1024),
    )(hidden_states, w_tiles, bias, mh, mwT)


# speedup vs baseline: 1.5861x; 1.3529x over previous
import functools

import numpy as np
import jax
import jax.numpy as jnp
from jax import lax
from jax.experimental import pallas as pl
from jax.experimental.pallas import tpu as pltpu
from jax.experimental.shard_map import shard_map
from jax.sharding import Mesh, PartitionSpec

_BN_EPS = 1e-5


def _interp_matrix(n_in, n_out):
    m = np.zeros((n_out, n_in), dtype=np.float32)
    if n_in == 1:
        m[:, 0] = 1.0
        return m
    src = np.arange(n_out, dtype=np.float64) * (n_in - 1) / (n_out - 1)
    lo = np.clip(np.floor(src).astype(np.int64), 0, n_in - 2)
    frac = (src - lo).astype(np.float32)
    m[np.arange(n_out), lo] = 1.0 - frac
    m[np.arange(n_out), lo + 1] = frac
    return m


def _fused_decoder_kernel(x_ref, w_ref, b_ref, mh_ref, mwT_ref, o_ref, u_ref):
    h = mh_ref.shape[1]
    zt = lax.dot_general(
        w_ref[0], x_ref[0], (((1,), (1,)), ((), ())),
        preferred_element_type=jnp.float32)
    zt = jnp.maximum(zt + b_ref[0], 0.0)
    mwT = mwT_ref[...]
    for hh in range(h):
        u_ref[:, hh, :] = jnp.dot(
            zt[:, hh * h:(hh + 1) * h], mwT, preferred_element_type=jnp.float32)
    mh = mh_ref[...]
    for c in range(o_ref.shape[1]):
        o_ref[0, c] = jnp.dot(
            mh, u_ref[c], preferred_element_type=jnp.float32
        ).astype(o_ref.dtype)


def _choose_kt(K):
    for t in (64, 32, 16, 8, 4, 2, 1):
        if K % t == 0:
            return t
    return K


def _decode_shard(hs, w_tiles, bias, mh, mwT, *, kt, OH, OW):
    B, P, C = hs.shape
    G = w_tiles.shape[0]
    h = mh.shape[1]
    return pl.pallas_call(
        _fused_decoder_kernel,
        out_shape=jax.ShapeDtypeStruct((B, G * kt, OH, OW), hs.dtype),
        grid=(B, G),
        in_specs=[
            pl.BlockSpec((1, P, C), lambda b, g: (b, 0, 0)),
            pl.BlockSpec((1, kt, C), lambda b, g: (g, 0, 0)),
            pl.BlockSpec((1, kt, 1), lambda b, g: (g, 0, 0)),
            pl.BlockSpec((OH, h), lambda b, g: (0, 0)),
            pl.BlockSpec((h, OW), lambda b, g: (0, 0)),
        ],
        out_specs=pl.BlockSpec((1, kt, OH, OW), lambda b, g: (b, g, 0, 0)),
        scratch_shapes=[pltpu.VMEM((kt, h, OW), jnp.float32)],
        compiler_params=pltpu.CompilerParams(
            dimension_semantics=("arbitrary", "arbitrary"),
            vmem_limit_bytes=60 * 1024 * 1024),
    )(hs, w_tiles, bias, mh, mwT)


def kernel(w, gamma, beta, mean, var, hidden_states):
    B, P, C = hidden_states.shape
    h = int(round(np.sqrt(P)))
    assert h * h == P
    K = w.shape[0]
    OH, OW = 224, 224

    kt = _choose_kt(K)
    G = K // kt
    scale = gamma / jnp.sqrt(var + _BN_EPS)
    wf = (w * scale[:, None]).astype(jnp.float32)
    w_tiles = wf.reshape(G, kt, C)
    bias = (beta - mean * scale).reshape(G, kt, 1).astype(jnp.float32)
    mh = jnp.asarray(_interp_matrix(h, OH))
    mwT = jnp.asarray(_interp_matrix(h, OW).T)

    run = functools.partial(_decode_shard, kt=kt, OH=OH, OW=OW)

    devs = jax.devices()
    nd = len(devs)
    while nd > 1 and B % nd != 0:
        nd -= 1
    if nd > 1:
        mesh = Mesh(np.array(devs[:nd]), ("d",))
        rep = PartitionSpec()
        specs = dict(
            in_specs=(PartitionSpec("d"), rep, rep, rep, rep),
            out_specs=PartitionSpec("d"),
        )
        try:
            run = shard_map(run, mesh=mesh, check_vma=False, **specs)
        except TypeError:
            run = shard_map(run, mesh=mesh, check_rep=False, **specs)
    return run(hidden_states, w_tiles, bias, mh, mwT)
